# baseline (speedup 1.0000x reference)
import functools

import jax
import jax.numpy as jnp
from jax import lax
from jax.experimental import pallas as pl
from jax.experimental.pallas import tpu as pltpu
from jax.experimental.pallas import tpu_sc as plsc

NC = 2
NS = 16
NW = NC * NS
LANES = 16

K = 128
NBUF = 3
ZCH = 80
ROWS_MAIN = 640


def _sc_body(n_nodes, nb, d,
             x_hbm, row_hbm, col_hbm, w_hbm, out_hbm,
             acc, colvs, rowvs, wvs, msgss, csems, rwsems, gsems, wsems,
             dsem):
  cid = lax.axis_index("c")
  sid = lax.axis_index("s")
  wid = sid * NC + cid
  nvec = d // LANES
  nfull = nb // NW
  nbt = nfull + jnp.where(nfull * NW + wid < nb, 1, 0)

  zvec = jnp.zeros((LANES,), jnp.float32)
  zbuf = msgss[0]

  def zfill(i, _):
    for j in range(nvec):
      zbuf[i, pl.ds(j * LANES, LANES)] = zvec
    return 0
  lax.fori_loop(0, ZCH, zfill, 0)

  rows_t = jnp.where(sid < NS - 1, ROWS_MAIN, n_nodes - ROWS_MAIN * (NS - 1))
  rbase = sid * ROWS_MAIN

  def zstart(i, _):
    pltpu.async_copy(zbuf.at[pl.ds(0, ZCH)],
                     acc.at[pl.ds(rbase + i * ZCH, ZCH)], dsem)
    return 0

  def zwait(i, _):
    pltpu.make_async_copy(zbuf.at[pl.ds(0, ZCH)],
                          acc.at[pl.ds(rbase + i * ZCH, ZCH)], dsem).wait()
    return 0

  with jax.named_scope("sc_init"):
    lax.fori_loop(0, rows_t // ZCH, zstart, 0)
    lax.fori_loop(0, rows_t // ZCH, zwait, 0)
    plsc.subcore_barrier()

  def eoff(t):
    return pl.multiple_of((wid + NW * t) * K, K)

  def start_col(t, s):
    pltpu.async_copy(col_hbm.at[pl.ds(eoff(t), K)], colvs[s], csems[s])

  def wait_col(t, s):
    pltpu.make_async_copy(col_hbm.at[pl.ds(eoff(t), K)], colvs[s],
                          csems[s]).wait()

  def start_roww(t, s):
    pltpu.async_copy(row_hbm.at[pl.ds(eoff(t), K)], rowvs[s], rwsems[s])
    pltpu.async_copy(w_hbm.at[pl.ds(eoff(t), K)], wvs[s], rwsems[s])

  def wait_roww(t, s):
    pltpu.make_async_copy(row_hbm.at[pl.ds(eoff(t), K)], rowvs[s],
                          rwsems[s]).wait()
    pltpu.make_async_copy(w_hbm.at[pl.ds(eoff(t), K)], wvs[s],
                          rwsems[s]).wait()

  def start_gather(s):
    pltpu.async_copy(x_hbm.at[colvs[s]], msgss[s], gsems[s])

  def wait_gather(s):
    pltpu.make_async_copy(x_hbm.at[colvs[s]], msgss[s], gsems[s]).wait()

  def start_scatter(s):
    pltpu.async_copy(msgss[s], acc.at[rowvs[s]], wsems[s], add=True)

  def wait_scatter(s):
    pltpu.make_async_copy(msgss[s], acc.at[rowvs[s]], wsems[s]).wait()

  def scale(s):
    wv = wvs[s]
    msgs = msgss[s]

    def sbody(g, _):
      w16 = wv[pl.ds(g * LANES, LANES)]
      for q in range(LANES):
        k = g * LANES + q
        wspl = w16.at[jnp.full((LANES,), q, jnp.int32)].get(
            mode="promise_in_bounds")
        for j in range(nvec):
          sl = (k, pl.ds(j * LANES, LANES))
          msgs[sl] = msgs[sl] * wspl
      return 0
    lax.fori_loop(0, K // LANES, sbody, 0)

  def unit(t, u):
    s = u % NBUF
    s1 = (u + 1) % NBUF
    s2 = (u + 2) % NBUF

    @pl.when(t >= 2)
    def _():
      wait_scatter(s1)

    @pl.when(t + 2 < nbt)
    def _():
      start_col(t + 2, s2)

    @pl.when(t + 1 < nbt)
    def _():
      start_roww(t + 1, s1)
      wait_col(t + 1, s1)
      start_gather(s1)
    wait_gather(s)
    wait_roww(t, s)
    scale(s)
    start_scatter(s)

  with jax.named_scope("sc_edges"):
    start_col(0, 0)
    start_col(1, 1)
    start_roww(0, 0)
    wait_col(0, 0)
    start_gather(0)

    def triple(i, _):
      t0 = NBUF * i
      for u in range(NBUF):
        unit(t0 + u, u)
      return 0
    lax.fori_loop(0, nfull // NBUF, triple, 0)

    @pl.when(nbt > nfull)
    def _():
      unit(nfull, nfull % NBUF)
      wait_scatter(nfull % NBUF)

    @pl.when(nbt == nfull)
    def _():
      wait_scatter((nfull - 2) % NBUF)
    wait_scatter((nfull - 1) % NBUF)

    plsc.subcore_barrier()

  def dstart(i, _):
    pltpu.async_copy(acc.at[pl.ds(rbase + i * ZCH, ZCH)],
                     out_hbm.at[cid, pl.ds(rbase + i * ZCH, ZCH)], dsem)
    return 0

  def dwait(i, _):
    pltpu.make_async_copy(acc.at[pl.ds(rbase + i * ZCH, ZCH)],
                          out_hbm.at[cid, pl.ds(rbase + i * ZCH, ZCH)],
                          dsem).wait()
    return 0

  with jax.named_scope("sc_drain"):
    lax.fori_loop(0, rows_t // ZCH, dstart, 0)
    lax.fori_loop(0, rows_t // ZCH, dwait, 0)


def _sc_scatter(x, row, col, edge_weight):
  n, d = x.shape
  nb = edge_weight.shape[0] // K
  mesh = plsc.VectorSubcoreMesh(core_axis_name="c", subcore_axis_name="s")
  body = functools.partial(_sc_body, n, nb, d)
  return pl.kernel(
      body,
      out_type=jax.ShapeDtypeStruct((NC, n, d), jnp.float32),
      mesh=mesh,
      compiler_params=pltpu.CompilerParams(needs_layout_passes=False),
      scratch_types=[
          pltpu.VMEM_SHARED((n, d), jnp.float32),
          [pltpu.VMEM((K,), jnp.int32)] * NBUF,
          [pltpu.VMEM((K,), jnp.int32)] * NBUF,
          [pltpu.VMEM((K,), jnp.float32)] * NBUF,
          [pltpu.VMEM((K, d), jnp.float32)] * NBUF,
          [pltpu.SemaphoreType.DMA] * NBUF,
          [pltpu.SemaphoreType.DMA] * NBUF,
          [pltpu.SemaphoreType.DMA] * NBUF,
          [pltpu.SemaphoreType.DMA] * NBUF,
          pltpu.SemaphoreType.DMA,
      ],
  )(x, row, col, edge_weight)


def _tc_linear_body(p0_ref, p1_ref, w_ref, b_ref, o_ref):
  acc = p0_ref[0] + p1_ref[0]
  y = lax.dot_general(acc, w_ref[...], (((1,), (1,)), ((), ())),
                      preferred_element_type=jnp.float32)
  o_ref[...] = y + b_ref[...]


def _tc_linear(partials, W, b):
  _, n, d = partials.shape
  blk = 1000
  grid = (n // blk,)
  return pl.pallas_call(
      _tc_linear_body,
      grid=grid,
      in_specs=[
          pl.BlockSpec((1, blk, d), lambda i: (0, i, 0)),
          pl.BlockSpec((1, blk, d), lambda i: (1, i, 0)),
          pl.BlockSpec((d, d), lambda i: (0, 0)),
          pl.BlockSpec((1, d), lambda i: (0, 0)),
      ],
      out_specs=pl.BlockSpec((blk, d), lambda i: (i, 0)),
      out_shape=jax.ShapeDtypeStruct((n, d), jnp.float32),
  )(partials, partials, W, b[None, :])


@jax.jit
def kernel(x, edge_index, edge_weight, W, b):
  partials = _sc_scatter(x, edge_index[0].astype(jnp.int32),
                         edge_index[1].astype(jnp.int32),
                         edge_weight.astype(jnp.float32))
  return _tc_linear(partials, W, b)


# speedup vs baseline: 1.0464x; 1.0464x over previous
import functools

import jax
import jax.numpy as jnp
from jax import lax
from jax.experimental import pallas as pl
from jax.experimental.pallas import tpu as pltpu
from jax.experimental.pallas import tpu_sc as plsc

NC = 2
NS = 16
NW = NC * NS
LANES = 16

K = 128
MBUF = 3
EBUF = 4
ZCH = 80
ROWS_MAIN = 640
ROW_I = 0
COL_I = 1


def _sc_body(n_nodes, nb, d,
             x_hbm, ei_hbm, w_hbm, out_hbm,
             acc, eivs, wvs, msgss, esems, wsems4, gsems, ssems, dsem):
  cid = lax.axis_index("c")
  sid = lax.axis_index("s")
  wid = sid * NC + cid
  nvec = d // LANES
  nfull = nb // NW
  nbt = nfull + jnp.where(nfull * NW + wid < nb, 1, 0)

  zvec = jnp.zeros((LANES,), jnp.float32)
  zbuf = msgss[0]

  def zfill(i, _):
    for j in range(nvec):
      zbuf[i, pl.ds(j * LANES, LANES)] = zvec
    return 0
  lax.fori_loop(0, ZCH, zfill, 0)

  rows_t = jnp.where(sid < NS - 1, ROWS_MAIN, n_nodes - ROWS_MAIN * (NS - 1))
  rbase = sid * ROWS_MAIN

  def zstart(i, _):
    pltpu.async_copy(zbuf.at[pl.ds(0, ZCH)],
                     acc.at[pl.ds(rbase + i * ZCH, ZCH)], dsem)
    return 0

  def zwait(i, _):
    pltpu.make_async_copy(zbuf.at[pl.ds(0, ZCH)],
                          acc.at[pl.ds(rbase + i * ZCH, ZCH)], dsem).wait()
    return 0

  with jax.named_scope("sc_init"):
    lax.fori_loop(0, rows_t // ZCH, zstart, 0)
    lax.fori_loop(0, rows_t // ZCH, zwait, 0)
    plsc.subcore_barrier()

  def eoff(t):
    return pl.multiple_of((wid + NW * t) * K, K)

  def start_ei(t, se):
    pltpu.async_copy(ei_hbm.at[:, pl.ds(eoff(t), K)], eivs[se], esems[se])
    pltpu.async_copy(w_hbm.at[pl.ds(eoff(t), K)], wvs[se], wsems4[se])

  def wait_ei(t, se):
    pltpu.make_async_copy(ei_hbm.at[:, pl.ds(eoff(t), K)], eivs[se],
                          esems[se]).wait()

  def wait_w(t, se):
    pltpu.make_async_copy(w_hbm.at[pl.ds(eoff(t), K)], wvs[se],
                          wsems4[se]).wait()

  def start_gather(sm, se):
    pltpu.async_copy(x_hbm.at[eivs[se].at[COL_I]], msgss[sm], gsems[sm])

  def wait_gather(sm, se):
    pltpu.make_async_copy(x_hbm.at[eivs[se].at[COL_I]], msgss[sm],
                          gsems[sm]).wait()

  def start_scatter(sm, se):
    pltpu.async_copy(msgss[sm], acc.at[eivs[se].at[ROW_I]], ssems[sm],
                     add=True)

  def wait_scatter(sm, se):
    pltpu.make_async_copy(msgss[sm], acc.at[eivs[se].at[ROW_I]],
                          ssems[sm]).wait()

  def scale(sm, se):
    wv = wvs[se]
    msgs = msgss[sm]

    def sbody(g, _):
      w16 = wv[pl.ds(g * LANES, LANES)]
      for q in range(LANES):
        k = g * LANES + q
        wspl = w16.at[jnp.full((LANES,), q, jnp.int32)].get(
            mode="promise_in_bounds")
        for j in range(nvec):
          sl = (k, pl.ds(j * LANES, LANES))
          msgs[sl] = msgs[sl] * wspl
      return 0
    lax.fori_loop(0, K // LANES, sbody, 0)

  def unit(t, um, ue):
    sm = um % MBUF
    sm1 = (um + 1) % MBUF
    se = ue % EBUF
    se1 = (ue + 1) % EBUF
    se2 = (ue + 2) % EBUF

    @pl.when(t >= 2)
    def _():
      wait_scatter(sm1, se2)

    @pl.when(t + 2 < nbt)
    def _():
      start_ei(t + 2, se2)

    @pl.when(t + 1 < nbt)
    def _():
      wait_ei(t + 1, se1)
      start_gather(sm1, se1)
    wait_gather(sm, se)
    wait_w(t, se)
    scale(sm, se)
    start_scatter(sm, se)

  with jax.named_scope("sc_edges"):
    start_ei(0, 0)
    start_ei(1, 1)
    wait_ei(0, 0)
    start_gather(0, 0)

    def l12(i, _):
      t0 = 12 * i
      for u in range(12):
        unit(t0 + u, u % MBUF, u % EBUF)
      return 0
    nl12 = nfull // 12
    lax.fori_loop(0, nl12, l12, 0)
    for u in range(12 * nl12, nfull):
      unit(u, u % MBUF, u % EBUF)

    @pl.when(nbt > nfull)
    def _():
      unit(nfull, nfull % MBUF, nfull % EBUF)
      wait_scatter(nfull % MBUF, nfull % EBUF)

    @pl.when(nbt == nfull)
    def _():
      wait_scatter((nfull - 2) % MBUF, (nfull - 2) % EBUF)
    wait_scatter((nfull - 1) % MBUF, (nfull - 1) % EBUF)

    plsc.subcore_barrier()

  def dstart(i, _):
    pltpu.async_copy(acc.at[pl.ds(rbase + i * ZCH, ZCH)],
                     out_hbm.at[cid, pl.ds(rbase + i * ZCH, ZCH)], dsem)
    return 0

  def dwait(i, _):
    pltpu.make_async_copy(acc.at[pl.ds(rbase + i * ZCH, ZCH)],
                          out_hbm.at[cid, pl.ds(rbase + i * ZCH, ZCH)],
                          dsem).wait()
    return 0

  with jax.named_scope("sc_drain"):
    lax.fori_loop(0, rows_t // ZCH, dstart, 0)
    lax.fori_loop(0, rows_t // ZCH, dwait, 0)


def _sc_scatter(x, edge_index, edge_weight):
  n, d = x.shape
  nb = edge_weight.shape[0] // K
  mesh = plsc.VectorSubcoreMesh(core_axis_name="c", subcore_axis_name="s")
  body = functools.partial(_sc_body, n, nb, d)
  return pl.kernel(
      body,
      out_type=jax.ShapeDtypeStruct((NC, n, d), jnp.float32),
      mesh=mesh,
      compiler_params=pltpu.CompilerParams(needs_layout_passes=False),
      scratch_types=[
          pltpu.VMEM_SHARED((n, d), jnp.float32),
          [pltpu.VMEM((2, K), jnp.int32)] * EBUF,
          [pltpu.VMEM((K,), jnp.float32)] * EBUF,
          [pltpu.VMEM((K, d), jnp.float32)] * MBUF,
          [pltpu.SemaphoreType.DMA] * EBUF,
          [pltpu.SemaphoreType.DMA] * EBUF,
          [pltpu.SemaphoreType.DMA] * MBUF,
          [pltpu.SemaphoreType.DMA] * MBUF,
          pltpu.SemaphoreType.DMA,
      ],
  )(x, edge_index, edge_weight)


def _tc_linear_body(p0_ref, p1_ref, w_ref, b_ref, o_ref):
  acc = p0_ref[0] + p1_ref[0]
  y = lax.dot_general(acc, w_ref[...], (((1,), (1,)), ((), ())),
                      preferred_element_type=jnp.float32)
  o_ref[...] = y + b_ref[...]


def _tc_linear(partials, W, b):
  _, n, d = partials.shape
  blk = 1000
  grid = (n // blk,)
  return pl.pallas_call(
      _tc_linear_body,
      grid=grid,
      in_specs=[
          pl.BlockSpec((1, blk, d), lambda i: (0, i, 0)),
          pl.BlockSpec((1, blk, d), lambda i: (1, i, 0)),
          pl.BlockSpec((d, d), lambda i: (0, 0)),
          pl.BlockSpec((1, d), lambda i: (0, 0)),
      ],
      out_specs=pl.BlockSpec((blk, d), lambda i: (i, 0)),
      out_shape=jax.ShapeDtypeStruct((n, d), jnp.float32),
  )(partials, partials, W, b[None, :])


@jax.jit
def kernel(x, edge_index, edge_weight, W, b):
  partials = _sc_scatter(x, edge_index.astype(jnp.int32),
                         edge_weight.astype(jnp.float32))
  return _tc_linear(partials, W, b)


# speedup vs baseline: 1.0647x; 1.0175x over previous
import functools

import jax
import jax.numpy as jnp
from jax import lax
from jax.experimental import pallas as pl
from jax.experimental.pallas import tpu as pltpu
from jax.experimental.pallas import tpu_sc as plsc

NC = 2
NS = 16
NW = NC * NS
LANES = 16

K = 128
MBUF = 3
EBUF = 4
ZCH = 80
ROWS_MAIN = 640
ROW_I = 0
COL_I = 1


def _sc_body(n_nodes, nb, d,
             x_hbm, ei_hbm, w_hbm, out_hbm,
             acc, eivs, wvs, msgss, esems, wsems4, gsems, ssems, dsem):
  cid = lax.axis_index("c")
  sid = lax.axis_index("s")
  wid = sid * NC + cid
  nvec = d // LANES
  nfull = nb // NW
  nbt = nfull + jnp.where(nfull * NW + wid < nb, 1, 0)

  zvec = jnp.zeros((LANES,), jnp.float32)
  zbuf = msgss[0]

  def zfill(i, _):
    for j in range(nvec):
      zbuf[i, pl.ds(j * LANES, LANES)] = zvec
    return 0
  lax.fori_loop(0, ZCH, zfill, 0)

  rows_t = jnp.where(sid < NS - 1, ROWS_MAIN, n_nodes - ROWS_MAIN * (NS - 1))
  rbase = sid * ROWS_MAIN

  def zstart(i, _):
    pltpu.async_copy(zbuf.at[pl.ds(0, ZCH)],
                     acc.at[pl.ds(rbase + i * ZCH, ZCH)], dsem)
    return 0

  def zwait(i, _):
    pltpu.make_async_copy(zbuf.at[pl.ds(0, ZCH)],
                          acc.at[pl.ds(rbase + i * ZCH, ZCH)], dsem).wait()
    return 0

  with jax.named_scope("sc_init"):
    lax.fori_loop(0, rows_t // ZCH, zstart, 0)
    lax.fori_loop(0, rows_t // ZCH, zwait, 0)
    plsc.subcore_barrier()

  def eoff(t):
    return pl.multiple_of((wid + NW * t) * K, K)

  def start_ei(t, se):
    pltpu.async_copy(ei_hbm.at[:, pl.ds(eoff(t), K)], eivs[se], esems[se])
    pltpu.async_copy(w_hbm.at[pl.ds(eoff(t), K)], wvs[se], wsems4[se])

  def wait_ei(t, se):
    pltpu.make_async_copy(ei_hbm.at[:, pl.ds(eoff(t), K)], eivs[se],
                          esems[se]).wait()

  def wait_w(t, se):
    pltpu.make_async_copy(w_hbm.at[pl.ds(eoff(t), K)], wvs[se],
                          wsems4[se]).wait()

  def start_gather(sm, se):
    pltpu.async_copy(x_hbm.at[eivs[se].at[COL_I]], msgss[sm], gsems[sm])

  def wait_gather(sm, se):
    pltpu.make_async_copy(x_hbm.at[eivs[se].at[COL_I]], msgss[sm],
                          gsems[sm]).wait()

  def start_scatter(sm, se):
    pltpu.async_copy(msgss[sm], acc.at[eivs[se].at[ROW_I]], ssems[sm],
                     add=True)

  def wait_scatter(sm, se):
    pltpu.make_async_copy(msgss[sm], acc.at[eivs[se].at[ROW_I]],
                          ssems[sm]).wait()

  def scale(sm, se):
    wv = wvs[se]
    msgs = msgss[sm]

    def sbody(g, _):
      w16 = wv[pl.ds(g * LANES, LANES)]
      for q in range(LANES):
        k = g * LANES + q
        wspl = w16.at[jnp.full((LANES,), q, jnp.int32)].get(
            mode="promise_in_bounds")
        for j in range(nvec):
          sl = (k, pl.ds(j * LANES, LANES))
          msgs[sl] = msgs[sl] * wspl
      return 0
    lax.fori_loop(0, K // LANES, sbody, 0)

  def unit(t, um, ue):
    sm = um % MBUF
    sm1 = (um + 1) % MBUF
    se = ue % EBUF
    se1 = (ue + 1) % EBUF
    se2 = (ue + 2) % EBUF

    @pl.when(t >= 2)
    def _():
      wait_scatter(sm1, se2)

    @pl.when(t + 2 < nbt)
    def _():
      start_ei(t + 2, se2)

    @pl.when(t + 1 < nbt)
    def _():
      wait_ei(t + 1, se1)
      start_gather(sm1, se1)
    wait_gather(sm, se)
    wait_w(t, se)
    scale(sm, se)
    start_scatter(sm, se)

  with jax.named_scope("sc_edges"):
    start_ei(0, 0)
    start_ei(1, 1)
    wait_ei(0, 0)
    start_gather(0, 0)

    def l12(i, _):
      t0 = 12 * i
      for u in range(12):
        unit(t0 + u, u % MBUF, u % EBUF)
      return 0
    nl12 = nfull // 12
    lax.fori_loop(0, nl12, l12, 0)
    for u in range(12 * nl12, nfull):
      unit(u, u % MBUF, u % EBUF)

    @pl.when(nbt > nfull)
    def _():
      unit(nfull, nfull % MBUF, nfull % EBUF)
      wait_scatter(nfull % MBUF, nfull % EBUF)

    @pl.when(nbt == nfull)
    def _():
      wait_scatter((nfull - 2) % MBUF, (nfull - 2) % EBUF)
    wait_scatter((nfull - 1) % MBUF, (nfull - 1) % EBUF)

    plsc.subcore_barrier()

  def dstart(i, _):
    pltpu.async_copy(acc.at[pl.ds(rbase + i * ZCH, ZCH)],
                     out_hbm.at[cid, pl.ds(rbase + i * ZCH, ZCH)], dsem)
    return 0

  def dwait(i, _):
    pltpu.make_async_copy(acc.at[pl.ds(rbase + i * ZCH, ZCH)],
                          out_hbm.at[cid, pl.ds(rbase + i * ZCH, ZCH)],
                          dsem).wait()
    return 0

  with jax.named_scope("sc_drain"):
    lax.fori_loop(0, rows_t // ZCH, dstart, 0)
    lax.fori_loop(0, rows_t // ZCH, dwait, 0)


def _sc_scatter(x, edge_index, edge_weight):
  n, d = x.shape
  nb = edge_weight.shape[0] // K
  mesh = plsc.VectorSubcoreMesh(core_axis_name="c", subcore_axis_name="s")
  body = functools.partial(_sc_body, n, nb, d)
  return pl.kernel(
      body,
      out_type=jax.ShapeDtypeStruct((NC, n, d), jnp.float32),
      mesh=mesh,
      compiler_params=pltpu.CompilerParams(needs_layout_passes=False),
      scratch_types=[
          pltpu.VMEM_SHARED((n, d), jnp.float32),
          [pltpu.VMEM((2, K), jnp.int32)] * EBUF,
          [pltpu.VMEM((K,), jnp.float32)] * EBUF,
          [pltpu.VMEM((K, d), jnp.float32)] * MBUF,
          [pltpu.SemaphoreType.DMA] * EBUF,
          [pltpu.SemaphoreType.DMA] * EBUF,
          [pltpu.SemaphoreType.DMA] * MBUF,
          [pltpu.SemaphoreType.DMA] * MBUF,
          pltpu.SemaphoreType.DMA,
      ],
  )(x, edge_index, edge_weight)


def _tc_linear_body(p0_ref, p1_ref, w_ref, b_ref, o_ref):
  acc = p0_ref[0] + p1_ref[0]
  y = lax.dot_general(acc, w_ref[...], (((1,), (1,)), ((), ())),
                      preferred_element_type=jnp.float32)
  o_ref[...] = y + b_ref[...]


def _tc_linear(partials, W, b):
  _, n, d = partials.shape
  blk = 2000
  grid = (n // blk,)
  return pl.pallas_call(
      _tc_linear_body,
      grid=grid,
      in_specs=[
          pl.BlockSpec((1, blk, d), lambda i: (0, i, 0)),
          pl.BlockSpec((1, blk, d), lambda i: (1, i, 0)),
          pl.BlockSpec((d, d), lambda i: (0, 0)),
          pl.BlockSpec((1, d), lambda i: (0, 0)),
      ],
      out_specs=pl.BlockSpec((blk, d), lambda i: (i, 0)),
      out_shape=jax.ShapeDtypeStruct((n, d), jnp.float32),
  )(partials, partials, W, b[None, :])


@jax.jit
def kernel(x, edge_index, edge_weight, W, b):
  partials = _sc_scatter(x, edge_index.astype(jnp.int32),
                         edge_weight.astype(jnp.float32))
  return _tc_linear(partials, W, b)


# speedup vs baseline: 1.0806x; 1.0149x over previous
import functools

import jax
import jax.numpy as jnp
from jax import lax
from jax.experimental import pallas as pl
from jax.experimental.pallas import tpu as pltpu
from jax.experimental.pallas import tpu_sc as plsc

NC = 2
NS = 16
NW = NC * NS
LANES = 16

K = 128
MBUF = 3
EBUF = 4
ZCH = 80
ROWS_MAIN = 640
ROW_I = 0
COL_I = 1


def _sc_body(n_nodes, nb, d,
             x_hbm, ei_hbm, w_hbm, out_hbm,
             acc, eivs, wvs, msgss, esems, wsems4, gsems, ssems, dsem):
  cid = lax.axis_index("c")
  sid = lax.axis_index("s")
  wid = sid * NC + cid
  nvec = d // LANES
  nfull = nb // NW
  nbt = nfull + jnp.where(nfull * NW + wid < nb, 1, 0)

  zvec = jnp.zeros((LANES,), jnp.float32)
  zbuf = msgss[0]

  def zfill(i, _):
    for j in range(nvec):
      zbuf[i, pl.ds(j * LANES, LANES)] = zvec
    return 0
  lax.fori_loop(0, ZCH, zfill, 0)

  rows_t = jnp.where(sid < NS - 1, ROWS_MAIN, n_nodes - ROWS_MAIN * (NS - 1))
  rbase = sid * ROWS_MAIN

  def zstart(i, _):
    pltpu.async_copy(zbuf.at[pl.ds(0, ZCH)],
                     acc.at[pl.ds(rbase + i * ZCH, ZCH)], dsem)
    return 0

  def zwait(i, _):
    pltpu.make_async_copy(zbuf.at[pl.ds(0, ZCH)],
                          acc.at[pl.ds(rbase + i * ZCH, ZCH)], dsem).wait()
    return 0

  with jax.named_scope("sc_init"):
    lax.fori_loop(0, rows_t // ZCH, zstart, 0)
    lax.fori_loop(0, rows_t // ZCH, zwait, 0)
    plsc.subcore_barrier()

  def eoff(t):
    return pl.multiple_of((wid + NW * t) * K, K)

  def start_ei(t, se):
    pltpu.async_copy(ei_hbm.at[:, pl.ds(eoff(t), K)], eivs[se], esems[se])
    pltpu.async_copy(w_hbm.at[pl.ds(eoff(t), K)], wvs[se], wsems4[se])

  def wait_ei(t, se):
    pltpu.make_async_copy(ei_hbm.at[:, pl.ds(eoff(t), K)], eivs[se],
                          esems[se]).wait()

  def wait_w(t, se):
    pltpu.make_async_copy(w_hbm.at[pl.ds(eoff(t), K)], wvs[se],
                          wsems4[se]).wait()

  def start_gather(sm, se):
    pltpu.async_copy(x_hbm.at[eivs[se].at[COL_I]], msgss[sm], gsems[sm])

  def wait_gather(sm, se):
    pltpu.make_async_copy(x_hbm.at[eivs[se].at[COL_I]], msgss[sm],
                          gsems[sm]).wait()

  def start_scatter(sm, se):
    pltpu.async_copy(msgss[sm], acc.at[eivs[se].at[ROW_I]], ssems[sm],
                     add=True)

  def wait_scatter(sm, se):
    pltpu.make_async_copy(msgss[sm], acc.at[eivs[se].at[ROW_I]],
                          ssems[sm]).wait()

  def scale(sm, se):
    wv = wvs[se]
    msgs = msgss[sm]

    def sbody(g, _):
      w16 = wv[pl.ds(g * LANES, LANES)]
      for q in range(LANES):
        k = g * LANES + q
        wspl = w16.at[jnp.full((LANES,), q, jnp.int32)].get(
            mode="promise_in_bounds")
        for j in range(nvec):
          sl = (k, pl.ds(j * LANES, LANES))
          msgs[sl] = msgs[sl] * wspl
      return 0
    lax.fori_loop(0, K // LANES, sbody, 0)

  def unit(t, um, ue):
    sm = um % MBUF
    sm1 = (um + 1) % MBUF
    se = ue % EBUF
    se1 = (ue + 1) % EBUF
    se2 = (ue + 2) % EBUF

    @pl.when(t >= 2)
    def _():
      wait_scatter(sm1, se2)

    @pl.when(t + 2 < nbt)
    def _():
      start_ei(t + 2, se2)

    @pl.when(t + 1 < nbt)
    def _():
      wait_ei(t + 1, se1)
      start_gather(sm1, se1)
    wait_gather(sm, se)
    wait_w(t, se)
    scale(sm, se)
    start_scatter(sm, se)

  with jax.named_scope("sc_edges"):
    start_ei(0, 0)
    start_ei(1, 1)
    wait_ei(0, 0)
    start_gather(0, 0)

    def l12(i, _):
      t0 = 12 * i
      for u in range(12):
        unit(t0 + u, u % MBUF, u % EBUF)
      return 0
    nl12 = nfull // 12
    lax.fori_loop(0, nl12, l12, 0)
    for u in range(12 * nl12, nfull):
      unit(u, u % MBUF, u % EBUF)

    @pl.when(nbt > nfull)
    def _():
      unit(nfull, nfull % MBUF, nfull % EBUF)
      wait_scatter(nfull % MBUF, nfull % EBUF)

    @pl.when(nbt == nfull)
    def _():
      wait_scatter((nfull - 2) % MBUF, (nfull - 2) % EBUF)
    wait_scatter((nfull - 1) % MBUF, (nfull - 1) % EBUF)

    plsc.subcore_barrier()

  def dstart(i, _):
    pltpu.async_copy(acc.at[pl.ds(rbase + i * ZCH, ZCH)],
                     out_hbm.at[cid, pl.ds(rbase + i * ZCH, ZCH)], dsem)
    return 0

  def dwait(i, _):
    pltpu.make_async_copy(acc.at[pl.ds(rbase + i * ZCH, ZCH)],
                          out_hbm.at[cid, pl.ds(rbase + i * ZCH, ZCH)],
                          dsem).wait()
    return 0

  with jax.named_scope("sc_drain"):
    lax.fori_loop(0, rows_t // ZCH, dstart, 0)
    lax.fori_loop(0, rows_t // ZCH, dwait, 0)


def _sc_scatter(x, edge_index, edge_weight):
  n, d = x.shape
  nb = edge_weight.shape[0] // K
  mesh = plsc.VectorSubcoreMesh(core_axis_name="c", subcore_axis_name="s")
  body = functools.partial(_sc_body, n, nb, d)
  return pl.kernel(
      body,
      out_type=jax.ShapeDtypeStruct((NC, n, d), jnp.float32),
      mesh=mesh,
      compiler_params=pltpu.CompilerParams(needs_layout_passes=False),
      scratch_types=[
          pltpu.VMEM_SHARED((n, d), jnp.float32),
          [pltpu.VMEM((2, K), jnp.int32)] * EBUF,
          [pltpu.VMEM((K,), jnp.float32)] * EBUF,
          [pltpu.VMEM((K, d), jnp.float32)] * MBUF,
          [pltpu.SemaphoreType.DMA] * EBUF,
          [pltpu.SemaphoreType.DMA] * EBUF,
          [pltpu.SemaphoreType.DMA] * MBUF,
          [pltpu.SemaphoreType.DMA] * MBUF,
          pltpu.SemaphoreType.DMA,
      ],
  )(x, edge_index, edge_weight)


def _tc_linear_body(p0_ref, p1_ref, w_ref, b_ref, o_ref):
  acc = p0_ref[0] + p1_ref[0]
  y = lax.dot_general(acc, w_ref[...], (((1,), (1,)), ((), ())),
                      preferred_element_type=jnp.float32)
  o_ref[...] = y + b_ref[...]


def _tc_linear(partials, W, b):
  _, n, d = partials.shape
  blk = 5000
  grid = (n // blk,)
  return pl.pallas_call(
      _tc_linear_body,
      grid=grid,
      in_specs=[
          pl.BlockSpec((1, blk, d), lambda i: (0, i, 0)),
          pl.BlockSpec((1, blk, d), lambda i: (1, i, 0)),
          pl.BlockSpec((d, d), lambda i: (0, 0)),
          pl.BlockSpec((1, d), lambda i: (0, 0)),
      ],
      out_specs=pl.BlockSpec((blk, d), lambda i: (i, 0)),
      out_shape=jax.ShapeDtypeStruct((n, d), jnp.float32),
  )(partials, partials, W, b[None, :])


@jax.jit
def kernel(x, edge_index, edge_weight, W, b):
  partials = _sc_scatter(x, edge_index.astype(jnp.int32),
                         edge_weight.astype(jnp.float32))
  return _tc_linear(partials, W, b)
